# baseline (speedup 1.0000x reference)
import jax
import jax.numpy as jnp
from jax import lax
from jax.experimental import pallas as pl
from jax.experimental.pallas import tpu as pltpu

_ROWS = 4096
_COLS = 1024
_BLOCK_ROWS = 512
_GRID = _ROWS // _BLOCK_ROWS


def _body(c_ref, w_ref, o_ref):
    i = pl.program_id(0)

    @pl.when(i == 0)
    def _patch():
        a = c_ref[0, 9] / (c_ref[0, 10] * 400000.0) * 0.001
        b = c_ref[0, 11] / c_ref[0, 10]
        blk = w_ref[...]
        rows = lax.broadcasted_iota(jnp.int32, blk.shape, 0)
        cols = lax.broadcasted_iota(jnp.int32, blk.shape, 1)
        blk = jnp.where((rows == 0) & (cols == 0), a, blk)
        blk = jnp.where((rows == 0) & (cols == 1), b, blk)
        o_ref[...] = blk

    @pl.when(i != 0)
    def _copy():
        o_ref[...] = w_ref[...]


def kernel(y, w, c, t):
    w2 = w.reshape(_ROWS, _COLS)
    c2 = c.reshape(1, 14)
    out = pl.pallas_call(
        _body,
        grid=(_GRID,),
        in_specs=[
            pl.BlockSpec(memory_space=pltpu.SMEM),
            pl.BlockSpec((_BLOCK_ROWS, _COLS), lambda i: (i, 0)),
        ],
        out_specs=pl.BlockSpec((_BLOCK_ROWS, _COLS), lambda i: (i, 0)),
        out_shape=jax.ShapeDtypeStruct((_ROWS, _COLS), jnp.float32),
    )(c2, w2)
    return out.reshape(-1)
